# baseline (speedup 1.0000x reference)
import functools

import jax
import jax.numpy as jnp
from jax import lax
from jax.experimental import pallas as pl
from jax.experimental.pallas import tpu as pltpu
from jax.experimental.pallas import tpu_sc as plsc

N = 256
F = 256
OC = 64
KNN = 16
HID = (F * OC) // 2
BLK = 8

NSC_MROWS = 96
NSC = NSC_MROWS * OC
NTC = N * OC - NSC
TC_STEPS = NTC // (BLK * OC)

NTILES = 32
RPT = NSC // NTILES
GR = 8
NGRP = RPT // GR
HHID = HID // 2
KUNROLL = 8


def _sc_fold_kernel(w1_hbm, w2t_hbm, out_hbm, w2_v, buf0, buf1, c_v,
                    sem0, sem1):
    cid = lax.axis_index("c")
    sid = lax.axis_index("s")
    wid = sid * 2 + cid
    base = NTC + wid * RPT

    pltpu.sync_copy(w2t_hbm, w2_v)

    bufs = (buf0, buf1)
    sems = (sem0, sem1)

    def _start(grp, h):
        pltpu.async_copy(
            w1_hbm.at[pl.ds(base + grp * GR, GR), pl.ds(h * HHID, HHID)],
            bufs[h], sems[h])

    def _wait(h):
        pltpu.make_async_copy(
            w1_hbm.at[pl.ds(base, GR), pl.ds(0, HHID)],
            bufs[h], sems[h]).wait()

    _start(0, 0)
    _start(0, 1)

    lanes = lax.broadcasted_iota(jnp.int32, (16,), 0)
    zero = jnp.zeros((16,), jnp.float32)

    def _half(grp, h, accs):
        _wait(h)
        buf = bufs[h]

        def _kbody(k, accs):
            res = list(accs)
            for u in range(KUNROLL):
                loc = pl.multiple_of((k * KUNROLL + u) * 16, 16)
                off = pl.multiple_of(h * HHID + (k * KUNROLL + u) * 16, 16)
                b0 = w2_v[0, pl.ds(off, 16)]
                b1 = w2_v[1, pl.ds(off, 16)]
                for rr in range(GR):
                    a = buf[rr, pl.ds(loc, 16)]
                    res[2 * rr] = res[2 * rr] + a * b0
                    res[2 * rr + 1] = res[2 * rr + 1] + a * b1
            return tuple(res)

        accs = lax.fori_loop(0, HHID // (16 * KUNROLL), _kbody, accs)
        @pl.when(grp + 1 < NGRP)
        def _prefetch():
            _start(grp + 1, h)
        return accs

    def _outer(d, carry):
        for e in range(4):
            rvec0, rvec1 = zero, zero
            for gi in range(2):
                grp = d * 8 + e * 2 + gi
                accs = (zero,) * (2 * GR)
                accs = _half(grp, 0, accs)
                accs = _half(grp, 1, accs)
                for rr in range(GR):
                    lane_idx = gi * GR + rr
                    s0 = accs[2 * rr]
                    s1 = accs[2 * rr + 1]
                    for sh in (8, 4, 2, 1):
                        idx = jnp.bitwise_xor(lanes, sh)
                        s0 = s0 + s0.at[idx].get(mode="promise_in_bounds")
                        s1 = s1 + s1.at[idx].get(mode="promise_in_bounds")
                    rvec0 = jnp.where(lanes == lane_idx, s0, rvec0)
                    rvec1 = jnp.where(lanes == lane_idx, s1, rvec1)
            c_v[0, d, pl.ds(e * 16, 16)] = rvec0
            c_v[1, d, pl.ds(e * 16, 16)] = rvec1
        return carry

    lax.fori_loop(0, RPT // OC, _outer, 0)

    pltpu.sync_copy(c_v, out_hbm.at[wid])


def _sc_fold(W1, W2t):
    mesh = plsc.VectorSubcoreMesh(core_axis_name="c", subcore_axis_name="s")
    kfn = functools.partial(
        pl.kernel, mesh=mesh,
        out_type=jax.ShapeDtypeStruct((NTILES, 2, RPT // OC, OC), jnp.float32),
        scratch_types=[
            pltpu.VMEM((2, HID), jnp.float32),
            pltpu.VMEM((GR, HHID), jnp.float32),
            pltpu.VMEM((GR, HHID), jnp.float32),
            pltpu.VMEM((2, RPT // OC, OC), jnp.float32),
            pltpu.SemaphoreType.DMA,
            pltpu.SemaphoreType.DMA,
        ],
    )(_sc_fold_kernel)
    raw = kfn(W1, W2t)
    return raw.transpose(1, 0, 2, 3).reshape(2, NSC_MROWS, OC)


def _select_topk(D, iota_j, k):
    onehots = []
    for _ in range(k):
        rowmin = jnp.min(D, axis=1, keepdims=True)
        cand = jnp.where(D == rowmin, iota_j, N)
        jsel = jnp.min(cand, axis=1, keepdims=True)
        sel = (iota_j == jsel)
        onehots.append(sel.astype(jnp.float32))
        D = jnp.where(sel, jnp.float32(2e38), D)
    return onehots


def _graph_phase(x_ref, we_ref, be_ref, wm_ref, bm_ref):
    X = x_ref[...]
    Wt = we_ref[:F, :]
    Wb = we_ref[F:, :]
    P = jnp.dot(X, Wt - Wb, preferred_element_type=jnp.float32)
    Q = jnp.dot(X, Wb, preferred_element_type=jnp.float32)

    iota_i = jax.lax.broadcasted_iota(jnp.int32, (N, N), 0)
    iota_j = jax.lax.broadcasted_iota(jnp.int32, (N, N), 1)
    eye = iota_i == iota_j

    sq = jnp.sum(X * X, axis=1, keepdims=True)
    G = jax.lax.dot_general(X, X, (((1,), (1,)), ((), ())),
                            preferred_element_type=jnp.float32)
    D = sq + jnp.transpose(sq, (1, 0)) - 2.0 * G
    D = jnp.where(eye, D + 1e10, D)
    M = jnp.full((N, OC), -3e38, jnp.float32)
    for onehot in _select_topk(D, iota_j, KNN):
        selq = jnp.dot(onehot, Q, preferred_element_type=jnp.float32)
        M = jnp.maximum(M, selq)
    feats = jax.nn.relu(P + M + be_ref[...])

    sq2 = jnp.sum(feats * feats, axis=1, keepdims=True)
    G2 = jax.lax.dot_general(feats, feats, (((1,), (1,)), ((), ())),
                             preferred_element_type=jnp.float32)
    D2 = sq2 + jnp.transpose(sq2, (1, 0)) - 2.0 * G2
    D2 = jnp.where(eye, D2 + 1e10, D2)
    msum = jnp.zeros((N, N), jnp.float32)
    for onehot in _select_topk(D2, iota_j, KNN):
        msum = msum + onehot
    agg = jnp.dot(msum, feats, preferred_element_type=jnp.float32) * (1.0 / KNN)
    market = jnp.dot(agg, wm_ref[...], preferred_element_type=jnp.float32)
    return jax.nn.relu(market + bm_ref[...])


def _tc_main_kernel(x_ref, we_ref, be_ref, wm_ref, bm_ref, w1_ref, b1_ref,
                    w2t_ref, b2_ref, part_ref, mkt_ref, acc_ref):
    i = pl.program_id(0)

    @pl.when(i == 0)
    def _graph():
        mkt_ref[...] = _graph_phase(x_ref, we_ref, be_ref, wm_ref, bm_ref)
        acc_ref[...] = jnp.zeros((BLK, HID), jnp.float32)

    m = mkt_ref[pl.ds(i * BLK, BLK), :]
    w = w1_ref[...]
    acc_ref[...] = acc_ref[...] + jnp.sum(m[:, :, None] * w, axis=1)

    @pl.when(i == pl.num_programs(0) - 1)
    def _finish():
        h1 = jnp.sum(acc_ref[...], axis=0, keepdims=True) + b1_ref[...]
        w2t = w2t_ref[...]
        l0 = jnp.sum(h1 * w2t[0:1, :], axis=1, keepdims=True)
        l1 = jnp.sum(h1 * w2t[1:2, :], axis=1, keepdims=True)
        lane = jax.lax.broadcasted_iota(jnp.int32, (1, 2), 1)
        part_ref[...] = jnp.where(lane == 0, l0, l1) + b2_ref[...]


def _combine_kernel(part_ref, ct_ref, mkt_ref, out_ref):
    msc = mkt_ref[N - NSC_MROWS:, :]
    c0 = ct_ref[0]
    c1 = ct_ref[1]
    t0 = jnp.sum(jnp.sum(msc * c0, axis=1, keepdims=True),
                 axis=0, keepdims=True)
    t1 = jnp.sum(jnp.sum(msc * c1, axis=1, keepdims=True),
                 axis=0, keepdims=True)
    lane = jax.lax.broadcasted_iota(jnp.int32, (1, 2), 1)
    logits = part_ref[...] + jnp.where(lane == 0, t0, t1)
    mx = jnp.max(logits, axis=1, keepdims=True)
    e = jnp.exp(logits - mx)
    out_ref[...] = e / jnp.sum(e, axis=1, keepdims=True)


def kernel(X, W_edge, b_edge, W_mkt, b_mkt, W1, b1, W2, b2):
    W2t = W2.T
    cT = _sc_fold(W1, W2t)

    W1r = W1.reshape(N, OC, HID)
    partial, market = pl.pallas_call(
        _tc_main_kernel,
        grid=(TC_STEPS,),
        in_specs=[
            pl.BlockSpec((N, F), lambda i: (0, 0)),
            pl.BlockSpec((2 * F, OC), lambda i: (0, 0)),
            pl.BlockSpec((1, OC), lambda i: (0, 0)),
            pl.BlockSpec((OC, OC), lambda i: (0, 0)),
            pl.BlockSpec((1, OC), lambda i: (0, 0)),
            pl.BlockSpec((BLK, OC, HID), lambda i: (i, 0, 0)),
            pl.BlockSpec((1, HID), lambda i: (0, 0)),
            pl.BlockSpec((2, HID), lambda i: (0, 0)),
            pl.BlockSpec((1, 2), lambda i: (0, 0)),
        ],
        out_specs=[pl.BlockSpec((1, 2), lambda i: (0, 0)),
                   pl.BlockSpec((N, OC), lambda i: (0, 0))],
        out_shape=[jax.ShapeDtypeStruct((1, 2), jnp.float32),
                   jax.ShapeDtypeStruct((N, OC), jnp.float32)],
        scratch_shapes=[pltpu.VMEM((BLK, HID), jnp.float32)],
    )(X, W_edge, b_edge.reshape(1, OC), W_mkt, b_mkt.reshape(1, OC),
      W1r, b1.reshape(1, HID), W2t, b2.reshape(1, 2))

    probs = pl.pallas_call(
        _combine_kernel,
        out_shape=jax.ShapeDtypeStruct((1, 2), jnp.float32),
    )(partial, cT, market)
    return probs.reshape(2)


# speedup vs baseline: 1.1635x; 1.1635x over previous
import functools

import jax
import jax.numpy as jnp
from jax import lax
from jax.experimental import pallas as pl
from jax.experimental.pallas import tpu as pltpu
from jax.experimental.pallas import tpu_sc as plsc

N = 256
F = 256
OC = 64
KNN = 16
HID = (F * OC) // 2
BLK = 8

NSC_MROWS = 64
NSC = NSC_MROWS * OC
NTC = N * OC - NSC
TC_STEPS = NTC // (BLK * OC)

NTILES = 32
RPT = NSC // NTILES
GR = 8
NGRP = RPT // GR
HHID = HID // 2
KUNROLL = 8


def _sc_fold_kernel(w1_hbm, w2t_hbm, out_hbm, w2_v, buf0, buf1, c_v,
                    sem0, sem1):
    cid = lax.axis_index("c")
    sid = lax.axis_index("s")
    wid = sid * 2 + cid
    base = NTC + wid * RPT

    pltpu.sync_copy(w2t_hbm, w2_v)

    bufs = (buf0, buf1)
    sems = (sem0, sem1)

    def _start(grp, h):
        pltpu.async_copy(
            w1_hbm.at[pl.ds(base + grp * GR, GR), pl.ds(h * HHID, HHID)],
            bufs[h], sems[h])

    def _wait(h):
        pltpu.make_async_copy(
            w1_hbm.at[pl.ds(base, GR), pl.ds(0, HHID)],
            bufs[h], sems[h]).wait()

    _start(0, 0)
    _start(0, 1)

    lanes = lax.broadcasted_iota(jnp.int32, (16,), 0)
    zero = jnp.zeros((16,), jnp.float32)

    def _half(grp, h, accs):
        _wait(h)
        buf = bufs[h]

        def _kbody(k, accs):
            res = list(accs)
            for u in range(KUNROLL):
                loc = pl.multiple_of((k * KUNROLL + u) * 16, 16)
                off = pl.multiple_of(h * HHID + (k * KUNROLL + u) * 16, 16)
                b0 = w2_v[0, pl.ds(off, 16)]
                b1 = w2_v[1, pl.ds(off, 16)]
                for rr in range(GR):
                    a = buf[rr, pl.ds(loc, 16)]
                    res[2 * rr] = res[2 * rr] + a * b0
                    res[2 * rr + 1] = res[2 * rr + 1] + a * b1
            return tuple(res)

        accs = lax.fori_loop(0, HHID // (16 * KUNROLL), _kbody, accs)
        @pl.when(grp + 1 < NGRP)
        def _prefetch():
            _start(grp + 1, h)
        return accs

    def _outer(d, carry):
        for e in range(4):
            rvec0, rvec1 = zero, zero
            for gi in range(2):
                grp = d * 8 + e * 2 + gi
                accs = (zero,) * (2 * GR)
                accs = _half(grp, 0, accs)
                accs = _half(grp, 1, accs)
                for rr in range(GR):
                    lane_idx = gi * GR + rr
                    s0 = accs[2 * rr]
                    s1 = accs[2 * rr + 1]
                    for sh in (8, 4, 2, 1):
                        idx = jnp.bitwise_xor(lanes, sh)
                        s0 = s0 + s0.at[idx].get(mode="promise_in_bounds")
                        s1 = s1 + s1.at[idx].get(mode="promise_in_bounds")
                    rvec0 = jnp.where(lanes == lane_idx, s0, rvec0)
                    rvec1 = jnp.where(lanes == lane_idx, s1, rvec1)
            c_v[0, d, pl.ds(e * 16, 16)] = rvec0
            c_v[1, d, pl.ds(e * 16, 16)] = rvec1
        return carry

    lax.fori_loop(0, RPT // OC, _outer, 0)

    pltpu.sync_copy(c_v, out_hbm.at[wid])


def _sc_fold(W1, W2t):
    mesh = plsc.VectorSubcoreMesh(core_axis_name="c", subcore_axis_name="s")
    kfn = functools.partial(
        pl.kernel, mesh=mesh,
        out_type=jax.ShapeDtypeStruct((NTILES, 2, RPT // OC, OC), jnp.float32),
        scratch_types=[
            pltpu.VMEM((2, HID), jnp.float32),
            pltpu.VMEM((GR, HHID), jnp.float32),
            pltpu.VMEM((GR, HHID), jnp.float32),
            pltpu.VMEM((2, RPT // OC, OC), jnp.float32),
            pltpu.SemaphoreType.DMA,
            pltpu.SemaphoreType.DMA,
        ],
    )(_sc_fold_kernel)
    raw = kfn(W1, W2t)
    return raw.transpose(1, 0, 2, 3).reshape(2, NSC_MROWS, OC)


def _select_topk(D, iota_j, k):
    onehots = []
    for _ in range(k):
        rowmin = jnp.min(D, axis=1, keepdims=True)
        cand = jnp.where(D == rowmin, iota_j, N)
        jsel = jnp.min(cand, axis=1, keepdims=True)
        sel = (iota_j == jsel)
        onehots.append(sel.astype(jnp.float32))
        D = jnp.where(sel, jnp.float32(2e38), D)
    return onehots


def _graph_phase(x_ref, we_ref, be_ref, wm_ref, bm_ref):
    X = x_ref[...]
    Wt = we_ref[:F, :]
    Wb = we_ref[F:, :]
    P = jnp.dot(X, Wt - Wb, preferred_element_type=jnp.float32)
    Q = jnp.dot(X, Wb, preferred_element_type=jnp.float32)

    iota_i = jax.lax.broadcasted_iota(jnp.int32, (N, N), 0)
    iota_j = jax.lax.broadcasted_iota(jnp.int32, (N, N), 1)
    eye = iota_i == iota_j

    sq = jnp.sum(X * X, axis=1, keepdims=True)
    G = jax.lax.dot_general(X, X, (((1,), (1,)), ((), ())),
                            preferred_element_type=jnp.float32)
    D = sq + jnp.transpose(sq, (1, 0)) - 2.0 * G
    D = jnp.where(eye, D + 1e10, D)
    M = jnp.full((N, OC), -3e38, jnp.float32)
    for onehot in _select_topk(D, iota_j, KNN):
        selq = jnp.dot(onehot, Q, preferred_element_type=jnp.float32)
        M = jnp.maximum(M, selq)
    feats = jax.nn.relu(P + M + be_ref[...])

    sq2 = jnp.sum(feats * feats, axis=1, keepdims=True)
    G2 = jax.lax.dot_general(feats, feats, (((1,), (1,)), ((), ())),
                             preferred_element_type=jnp.float32)
    D2 = sq2 + jnp.transpose(sq2, (1, 0)) - 2.0 * G2
    D2 = jnp.where(eye, D2 + 1e10, D2)
    msum = jnp.zeros((N, N), jnp.float32)
    for onehot in _select_topk(D2, iota_j, KNN):
        msum = msum + onehot
    agg = jnp.dot(msum, feats, preferred_element_type=jnp.float32) * (1.0 / KNN)
    market = jnp.dot(agg, wm_ref[...], preferred_element_type=jnp.float32)
    return jax.nn.relu(market + bm_ref[...])


def _tc_main_kernel(x_ref, we_ref, be_ref, wm_ref, bm_ref, w1_ref, b1_ref,
                    w2t_ref, b2_ref, part_ref, mkt_ref, acc_ref):
    i = pl.program_id(0)

    @pl.when(i == 0)
    def _graph():
        mkt_ref[...] = _graph_phase(x_ref, we_ref, be_ref, wm_ref, bm_ref)
        acc_ref[...] = jnp.zeros((BLK, HID), jnp.float32)

    m = mkt_ref[pl.ds(i * BLK, BLK), :]
    w = w1_ref[...]
    acc_ref[...] = acc_ref[...] + jnp.sum(m[:, :, None] * w, axis=1)

    @pl.when(i == pl.num_programs(0) - 1)
    def _finish():
        h1 = jnp.sum(acc_ref[...], axis=0, keepdims=True) + b1_ref[...]
        w2t = w2t_ref[...]
        l0 = jnp.sum(h1 * w2t[0:1, :], axis=1, keepdims=True)
        l1 = jnp.sum(h1 * w2t[1:2, :], axis=1, keepdims=True)
        lane = jax.lax.broadcasted_iota(jnp.int32, (1, 2), 1)
        part_ref[...] = jnp.where(lane == 0, l0, l1) + b2_ref[...]


def _combine_kernel(part_ref, ct_ref, mkt_ref, out_ref):
    msc = mkt_ref[N - NSC_MROWS:, :]
    c0 = ct_ref[0]
    c1 = ct_ref[1]
    t0 = jnp.sum(jnp.sum(msc * c0, axis=1, keepdims=True),
                 axis=0, keepdims=True)
    t1 = jnp.sum(jnp.sum(msc * c1, axis=1, keepdims=True),
                 axis=0, keepdims=True)
    lane = jax.lax.broadcasted_iota(jnp.int32, (1, 2), 1)
    logits = part_ref[...] + jnp.where(lane == 0, t0, t1)
    mx = jnp.max(logits, axis=1, keepdims=True)
    e = jnp.exp(logits - mx)
    out_ref[...] = e / jnp.sum(e, axis=1, keepdims=True)


def kernel(X, W_edge, b_edge, W_mkt, b_mkt, W1, b1, W2, b2):
    W2t = W2.T
    cT = _sc_fold(W1, W2t)

    W1r = W1.reshape(N, OC, HID)
    partial, market = pl.pallas_call(
        _tc_main_kernel,
        grid=(TC_STEPS,),
        in_specs=[
            pl.BlockSpec((N, F), lambda i: (0, 0)),
            pl.BlockSpec((2 * F, OC), lambda i: (0, 0)),
            pl.BlockSpec((1, OC), lambda i: (0, 0)),
            pl.BlockSpec((OC, OC), lambda i: (0, 0)),
            pl.BlockSpec((1, OC), lambda i: (0, 0)),
            pl.BlockSpec((BLK, OC, HID), lambda i: (i, 0, 0)),
            pl.BlockSpec((1, HID), lambda i: (0, 0)),
            pl.BlockSpec((2, HID), lambda i: (0, 0)),
            pl.BlockSpec((1, 2), lambda i: (0, 0)),
        ],
        out_specs=[pl.BlockSpec((1, 2), lambda i: (0, 0)),
                   pl.BlockSpec((N, OC), lambda i: (0, 0))],
        out_shape=[jax.ShapeDtypeStruct((1, 2), jnp.float32),
                   jax.ShapeDtypeStruct((N, OC), jnp.float32)],
        scratch_shapes=[pltpu.VMEM((BLK, HID), jnp.float32)],
    )(X, W_edge, b_edge.reshape(1, OC), W_mkt, b_mkt.reshape(1, OC),
      W1r, b1.reshape(1, HID), W2t, b2.reshape(1, 2))

    probs = pl.pallas_call(
        _combine_kernel,
        out_shape=jax.ShapeDtypeStruct((1, 2), jnp.float32),
    )(partial, cT, market)
    return probs.reshape(2)


# speedup vs baseline: 1.1695x; 1.0051x over previous
import functools

import jax
import jax.numpy as jnp
from jax import lax
from jax.experimental import pallas as pl
from jax.experimental.pallas import tpu as pltpu
from jax.experimental.pallas import tpu_sc as plsc

N = 256
F = 256
OC = 64
KNN = 16
HID = (F * OC) // 2
BLK = 8

NSC_MROWS = 32
NSC = NSC_MROWS * OC
NTC = N * OC - NSC
TC_STEPS = NTC // (BLK * OC)

NTILES = 32
RPT = NSC // NTILES
GR = 8
NGRP = RPT // GR
HHID = HID // 2
KUNROLL = 8


def _sc_fold_kernel(w1_hbm, w2t_hbm, out_hbm, w2_v, buf0, buf1, c_v,
                    sem0, sem1):
    cid = lax.axis_index("c")
    sid = lax.axis_index("s")
    wid = sid * 2 + cid
    base = NTC + wid * RPT

    pltpu.sync_copy(w2t_hbm, w2_v)

    bufs = (buf0, buf1)
    sems = (sem0, sem1)

    def _start(grp, h):
        pltpu.async_copy(
            w1_hbm.at[pl.ds(base + grp * GR, GR), pl.ds(h * HHID, HHID)],
            bufs[h], sems[h])

    def _wait(h):
        pltpu.make_async_copy(
            w1_hbm.at[pl.ds(base, GR), pl.ds(0, HHID)],
            bufs[h], sems[h]).wait()

    _start(0, 0)
    _start(0, 1)

    lanes = lax.broadcasted_iota(jnp.int32, (16,), 0)
    zero = jnp.zeros((16,), jnp.float32)

    def _half(grp, h, accs):
        _wait(h)
        buf = bufs[h]

        def _kbody(k, accs):
            res = list(accs)
            for u in range(KUNROLL):
                loc = pl.multiple_of((k * KUNROLL + u) * 16, 16)
                off = pl.multiple_of(h * HHID + (k * KUNROLL + u) * 16, 16)
                b0 = w2_v[0, pl.ds(off, 16)]
                b1 = w2_v[1, pl.ds(off, 16)]
                for rr in range(GR):
                    a = buf[rr, pl.ds(loc, 16)]
                    res[2 * rr] = res[2 * rr] + a * b0
                    res[2 * rr + 1] = res[2 * rr + 1] + a * b1
            return tuple(res)

        accs = lax.fori_loop(0, HHID // (16 * KUNROLL), _kbody, accs)
        @pl.when(grp + 1 < NGRP)
        def _prefetch():
            _start(grp + 1, h)
        return accs

    def _outer(d, carry):
        for e in range(4):
            rvec0, rvec1 = zero, zero
            for gi in range(2):
                grp = d * 8 + e * 2 + gi
                accs = (zero,) * (2 * GR)
                accs = _half(grp, 0, accs)
                accs = _half(grp, 1, accs)
                for rr in range(GR):
                    lane_idx = gi * GR + rr
                    s0 = accs[2 * rr]
                    s1 = accs[2 * rr + 1]
                    for sh in (8, 4, 2, 1):
                        idx = jnp.bitwise_xor(lanes, sh)
                        s0 = s0 + s0.at[idx].get(mode="promise_in_bounds")
                        s1 = s1 + s1.at[idx].get(mode="promise_in_bounds")
                    rvec0 = jnp.where(lanes == lane_idx, s0, rvec0)
                    rvec1 = jnp.where(lanes == lane_idx, s1, rvec1)
            c_v[0, d, pl.ds(e * 16, 16)] = rvec0
            c_v[1, d, pl.ds(e * 16, 16)] = rvec1
        return carry

    lax.fori_loop(0, RPT // OC, _outer, 0)

    pltpu.sync_copy(c_v, out_hbm.at[wid])


def _sc_fold(W1, W2t):
    mesh = plsc.VectorSubcoreMesh(core_axis_name="c", subcore_axis_name="s")
    kfn = functools.partial(
        pl.kernel, mesh=mesh,
        out_type=jax.ShapeDtypeStruct((NTILES, 2, RPT // OC, OC), jnp.float32),
        scratch_types=[
            pltpu.VMEM((2, HID), jnp.float32),
            pltpu.VMEM((GR, HHID), jnp.float32),
            pltpu.VMEM((GR, HHID), jnp.float32),
            pltpu.VMEM((2, RPT // OC, OC), jnp.float32),
            pltpu.SemaphoreType.DMA,
            pltpu.SemaphoreType.DMA,
        ],
    )(_sc_fold_kernel)
    raw = kfn(W1, W2t)
    return raw.transpose(1, 0, 2, 3).reshape(2, NSC_MROWS, OC)


def _select_topk(D, iota_j, k):
    onehots = []
    for _ in range(k):
        rowmin = jnp.min(D, axis=1, keepdims=True)
        cand = jnp.where(D == rowmin, iota_j, N)
        jsel = jnp.min(cand, axis=1, keepdims=True)
        sel = (iota_j == jsel)
        onehots.append(sel.astype(jnp.float32))
        D = jnp.where(sel, jnp.float32(2e38), D)
    return onehots


def _graph_phase(x_ref, we_ref, be_ref, wm_ref, bm_ref):
    X = x_ref[...]
    Wt = we_ref[:F, :]
    Wb = we_ref[F:, :]
    P = jnp.dot(X, Wt - Wb, preferred_element_type=jnp.float32)
    Q = jnp.dot(X, Wb, preferred_element_type=jnp.float32)

    iota_i = jax.lax.broadcasted_iota(jnp.int32, (N, N), 0)
    iota_j = jax.lax.broadcasted_iota(jnp.int32, (N, N), 1)
    eye = iota_i == iota_j

    sq = jnp.sum(X * X, axis=1, keepdims=True)
    G = jax.lax.dot_general(X, X, (((1,), (1,)), ((), ())),
                            preferred_element_type=jnp.float32)
    D = sq + jnp.transpose(sq, (1, 0)) - 2.0 * G
    D = jnp.where(eye, D + 1e10, D)
    M = jnp.full((N, OC), -3e38, jnp.float32)
    for onehot in _select_topk(D, iota_j, KNN):
        selq = jnp.dot(onehot, Q, preferred_element_type=jnp.float32)
        M = jnp.maximum(M, selq)
    feats = jax.nn.relu(P + M + be_ref[...])

    sq2 = jnp.sum(feats * feats, axis=1, keepdims=True)
    G2 = jax.lax.dot_general(feats, feats, (((1,), (1,)), ((), ())),
                             preferred_element_type=jnp.float32)
    D2 = sq2 + jnp.transpose(sq2, (1, 0)) - 2.0 * G2
    D2 = jnp.where(eye, D2 + 1e10, D2)
    msum = jnp.zeros((N, N), jnp.float32)
    for onehot in _select_topk(D2, iota_j, KNN):
        msum = msum + onehot
    agg = jnp.dot(msum, feats, preferred_element_type=jnp.float32) * (1.0 / KNN)
    market = jnp.dot(agg, wm_ref[...], preferred_element_type=jnp.float32)
    return jax.nn.relu(market + bm_ref[...])


def _tc_main_kernel(x_ref, we_ref, be_ref, wm_ref, bm_ref, w1_ref, b1_ref,
                    w2t_ref, b2_ref, part_ref, mkt_ref, acc_ref):
    i = pl.program_id(0)

    @pl.when(i == 0)
    def _graph():
        mkt_ref[...] = _graph_phase(x_ref, we_ref, be_ref, wm_ref, bm_ref)
        acc_ref[...] = jnp.zeros((BLK, HID), jnp.float32)

    m = mkt_ref[pl.ds(i * BLK, BLK), :]
    w = w1_ref[...]
    acc_ref[...] = acc_ref[...] + jnp.sum(m[:, :, None] * w, axis=1)

    @pl.when(i == pl.num_programs(0) - 1)
    def _finish():
        h1 = jnp.sum(acc_ref[...], axis=0, keepdims=True) + b1_ref[...]
        w2t = w2t_ref[...]
        l0 = jnp.sum(h1 * w2t[0:1, :], axis=1, keepdims=True)
        l1 = jnp.sum(h1 * w2t[1:2, :], axis=1, keepdims=True)
        lane = jax.lax.broadcasted_iota(jnp.int32, (1, 2), 1)
        part_ref[...] = jnp.where(lane == 0, l0, l1) + b2_ref[...]


def _combine_kernel(part_ref, ct_ref, mkt_ref, out_ref):
    msc = mkt_ref[N - NSC_MROWS:, :]
    c0 = ct_ref[0]
    c1 = ct_ref[1]
    t0 = jnp.sum(jnp.sum(msc * c0, axis=1, keepdims=True),
                 axis=0, keepdims=True)
    t1 = jnp.sum(jnp.sum(msc * c1, axis=1, keepdims=True),
                 axis=0, keepdims=True)
    lane = jax.lax.broadcasted_iota(jnp.int32, (1, 2), 1)
    logits = part_ref[...] + jnp.where(lane == 0, t0, t1)
    mx = jnp.max(logits, axis=1, keepdims=True)
    e = jnp.exp(logits - mx)
    out_ref[...] = e / jnp.sum(e, axis=1, keepdims=True)


def kernel(X, W_edge, b_edge, W_mkt, b_mkt, W1, b1, W2, b2):
    W2t = W2.T
    cT = _sc_fold(W1, W2t)

    W1r = W1.reshape(N, OC, HID)
    partial, market = pl.pallas_call(
        _tc_main_kernel,
        grid=(TC_STEPS,),
        in_specs=[
            pl.BlockSpec((N, F), lambda i: (0, 0)),
            pl.BlockSpec((2 * F, OC), lambda i: (0, 0)),
            pl.BlockSpec((1, OC), lambda i: (0, 0)),
            pl.BlockSpec((OC, OC), lambda i: (0, 0)),
            pl.BlockSpec((1, OC), lambda i: (0, 0)),
            pl.BlockSpec((BLK, OC, HID), lambda i: (i, 0, 0)),
            pl.BlockSpec((1, HID), lambda i: (0, 0)),
            pl.BlockSpec((2, HID), lambda i: (0, 0)),
            pl.BlockSpec((1, 2), lambda i: (0, 0)),
        ],
        out_specs=[pl.BlockSpec((1, 2), lambda i: (0, 0)),
                   pl.BlockSpec((N, OC), lambda i: (0, 0))],
        out_shape=[jax.ShapeDtypeStruct((1, 2), jnp.float32),
                   jax.ShapeDtypeStruct((N, OC), jnp.float32)],
        scratch_shapes=[pltpu.VMEM((BLK, HID), jnp.float32)],
    )(X, W_edge, b_edge.reshape(1, OC), W_mkt, b_mkt.reshape(1, OC),
      W1r, b1.reshape(1, HID), W2t, b2.reshape(1, 2))

    probs = pl.pallas_call(
        _combine_kernel,
        out_shape=jax.ShapeDtypeStruct((1, 2), jnp.float32),
    )(partial, cT, market)
    return probs.reshape(2)


# speedup vs baseline: 1.2659x; 1.0825x over previous
import jax
import jax.numpy as jnp
from jax.experimental import pallas as pl
from jax.experimental.pallas import tpu as pltpu

N = 256
F = 256
OC = 64
KNN = 16
HID = (F * OC) // 2
BLK = 8


def _select_topk(D, iota_j, k):
    onehots = []
    for _ in range(k):
        rowmin = jnp.min(D, axis=1, keepdims=True)
        cand = jnp.where(D == rowmin, iota_j, N)
        jsel = jnp.min(cand, axis=1, keepdims=True)
        sel = (iota_j == jsel)
        onehots.append(sel.astype(jnp.float32))
        D = jnp.where(sel, jnp.float32(2e38), D)
    return onehots


def _graph_phase(x_ref, we_ref, be_ref, wm_ref, bm_ref):
    X = x_ref[...]

    iota_i = jax.lax.broadcasted_iota(jnp.int32, (N, N), 0)
    iota_j = jax.lax.broadcasted_iota(jnp.int32, (N, N), 1)
    eye = iota_i == iota_j

    sq = jnp.sum(X * X, axis=1, keepdims=True)
    G = jax.lax.dot_general(X, X, (((1,), (1,)), ((), ())),
                            precision=jax.lax.Precision.HIGHEST,
                            preferred_element_type=jnp.float32)
    D = sq + jnp.transpose(sq, (1, 0)) - 2.0 * G
    D = jnp.where(eye, D + 1e10, D)
    M = jnp.full((N, OC), -3e38, jnp.float32)
    for onehot in _select_topk(D, iota_j, KNN):
        xj = jnp.dot(onehot, X, preferred_element_type=jnp.float32)
        cc = jnp.concatenate([X, xj - X], axis=1)
        msg = jnp.dot(cc, we_ref[...],
                      precision=jax.lax.Precision.HIGHEST,
                      preferred_element_type=jnp.float32)
        M = jnp.maximum(M, msg)
    feats = jax.nn.relu(M + be_ref[...])

    sq2 = jnp.sum(feats * feats, axis=1, keepdims=True)
    G2 = jax.lax.dot_general(feats, feats, (((1,), (1,)), ((), ())),
                             precision=jax.lax.Precision.HIGHEST,
                             preferred_element_type=jnp.float32)
    D2 = sq2 + jnp.transpose(sq2, (1, 0)) - 2.0 * G2
    D2 = jnp.where(eye, D2 + 1e10, D2)
    msum = jnp.zeros((N, N), jnp.float32)
    for onehot in _select_topk(D2, iota_j, KNN):
        msum = msum + onehot
    agg = jnp.dot(msum, feats, preferred_element_type=jnp.float32) * (1.0 / KNN)
    market = jnp.dot(agg, wm_ref[...], preferred_element_type=jnp.float32)
    return jax.nn.relu(market + bm_ref[...])


def _fused_kernel(x_ref, we_ref, be_ref, wm_ref, bm_ref, w1_ref, b1_ref,
                  w2t_ref, b2_ref, out_ref, mkt_ref, acc_ref):
    i = pl.program_id(0)

    @pl.when(i == 0)
    def _graph():
        mkt_ref[...] = _graph_phase(x_ref, we_ref, be_ref, wm_ref, bm_ref)
        acc_ref[...] = jnp.zeros((BLK, HID), jnp.float32)

    m = mkt_ref[pl.ds(i * BLK, BLK), :]
    w = w1_ref[...]
    acc_ref[...] = acc_ref[...] + jnp.sum(m[:, :, None] * w, axis=1)

    @pl.when(i == pl.num_programs(0) - 1)
    def _finish():
        h1 = jnp.sum(acc_ref[...], axis=0, keepdims=True) + b1_ref[...]
        w2t = w2t_ref[...]
        l0 = jnp.sum(h1 * w2t[0:1, :], axis=1, keepdims=True)
        l1 = jnp.sum(h1 * w2t[1:2, :], axis=1, keepdims=True)
        lane = jax.lax.broadcasted_iota(jnp.int32, (1, 2), 1)
        logits = jnp.where(lane == 0, l0, l1) + b2_ref[...]
        mx = jnp.max(logits, axis=1, keepdims=True)
        e = jnp.exp(logits - mx)
        out_ref[...] = e / jnp.sum(e, axis=1, keepdims=True)


def kernel(X, W_edge, b_edge, W_mkt, b_mkt, W1, b1, W2, b2):
    W1r = W1.reshape(N, OC, HID)
    probs = pl.pallas_call(
        _fused_kernel,
        grid=(N // BLK,),
        in_specs=[
            pl.BlockSpec((N, F), lambda i: (0, 0)),
            pl.BlockSpec((2 * F, OC), lambda i: (0, 0)),
            pl.BlockSpec((1, OC), lambda i: (0, 0)),
            pl.BlockSpec((OC, OC), lambda i: (0, 0)),
            pl.BlockSpec((1, OC), lambda i: (0, 0)),
            pl.BlockSpec((BLK, OC, HID), lambda i: (i, 0, 0)),
            pl.BlockSpec((1, HID), lambda i: (0, 0)),
            pl.BlockSpec((2, HID), lambda i: (0, 0)),
            pl.BlockSpec((1, 2), lambda i: (0, 0)),
        ],
        out_specs=pl.BlockSpec((1, 2), lambda i: (0, 0)),
        out_shape=jax.ShapeDtypeStruct((1, 2), jnp.float32),
        scratch_shapes=[pltpu.VMEM((N, OC), jnp.float32),
                        pltpu.VMEM((BLK, HID), jnp.float32)],
    )(X, W_edge, b_edge.reshape(1, OC), W_mkt, b_mkt.reshape(1, OC),
      W1r, b1.reshape(1, HID), W2.T, b2.reshape(1, 2))
    return probs.reshape(2)
